# baseline (speedup 1.0000x reference)
import functools

import jax
import jax.numpy as jnp
from jax import lax
from jax.experimental import pallas as pl
from jax.experimental.pallas import tpu as pltpu
from jax.experimental.pallas import tpu_sc as plsc

VOCAB = 65536
EMBED = 32
LANES = 16
IDXB = 128
CHUNK = 1024
NB = CHUNK // IDXB

_info = plsc.get_sparse_core_info()
NUM_WORKERS = _info.num_cores * _info.num_subcores


def _encoder_body(table_hbm, idx_hbm, len_hbm, out_hbm,
                  idxraw_v, len_v, idxe_v, rows_v, sem):
    n_total = idx_hbm.shape[0]
    per_w = n_total // NUM_WORKERS
    n_chunks = per_w // CHUNK

    wid = lax.axis_index("s") * _info.num_cores + lax.axis_index("c")
    wbase = pl.multiple_of(wid * per_w, CHUNK)

    def chunk_body(c, carry):
        base = pl.multiple_of(wbase + c * CHUNK, CHUNK)
        pltpu.sync_copy(idx_hbm.at[pl.ds(base, CHUNK)], idxraw_v)
        pltpu.sync_copy(len_hbm.at[pl.ds(base, CHUNK)], len_v)

        for b in range(NB):
            def vec_body(i, carry2):
                off = b * IDXB + i * LANES
                idx16 = idxraw_v[pl.ds(off, LANES)]
                len16 = len_v[pl.ds(off, LANES)]
                idx16 = jnp.clip(idx16, 0, VOCAB - 1)
                eff = jnp.where(len16 > 0, idx16,
                                jnp.full((LANES,), VOCAB, jnp.int32))
                idxe_v[b, pl.ds(i * LANES, LANES)] = eff
                return carry2
            lax.fori_loop(0, IDXB // LANES, vec_body, 0)

        copies = [
            pltpu.make_async_copy(
                table_hbm.at[idxe_v.at[b]],
                rows_v.at[pl.ds(b * IDXB, IDXB)],
                sem,
            )
            for b in range(NB)
        ]
        for cp in copies:
            cp.start()
        for cp in copies:
            cp.wait()

        pltpu.sync_copy(rows_v, out_hbm.at[pl.ds(base, CHUNK)])
        return carry

    lax.fori_loop(0, n_chunks, chunk_body, 0)


def kernel(indices, lengths, table):
    b, l = indices.shape
    n = b * l
    idx_flat = indices.reshape(n)
    len_flat = lengths.reshape(n)
    table_ext = jnp.concatenate(
        [table, jnp.zeros((8, EMBED), jnp.float32)], axis=0)

    mesh = plsc.VectorSubcoreMesh(core_axis_name="c", subcore_axis_name="s")
    run = pl.kernel(
        _encoder_body,
        out_type=jax.ShapeDtypeStruct((n, EMBED), jnp.float32),
        mesh=mesh,
        scratch_types=[
            pltpu.VMEM((CHUNK,), jnp.int32),
            pltpu.VMEM((CHUNK,), jnp.int32),
            pltpu.VMEM((NB, IDXB), jnp.int32),
            pltpu.VMEM((CHUNK, EMBED), jnp.float32),
            pltpu.SemaphoreType.DMA,
        ],
        compiler_params=pltpu.CompilerParams(use_tc_tiling_on_sc=False),
    )
    out = run(table_ext, idx_flat, len_flat)
    return out.reshape(b, l, EMBED)
